# baseline (speedup 1.0000x reference)
import jax
import jax.numpy as jnp
from jax import lax
from jax.experimental import pallas as pl
from jax.experimental.pallas import tpu as pltpu
from jax.experimental.pallas import tpu_sc as plsc

N = 100000
D = 128
C = 10
CP = 16
S = 512
DS = S
SA = S + 8
NC = 2
NS = 16
NH = 2
GRIDH = 2
TPW = 25088
NPH = GRIDH * TPW
NP = NH * NPH
RQ = TPW // 8
PRH = GRIDH * RQ
WR = PRH // 4


def _mlp_body(x_ref, w1_ref, b1_ref, w2_ref, b2_ref, out_ref):
    h = jnp.dot(x_ref[...], w1_ref[...], preferred_element_type=jnp.float32)
    h = jnp.maximum(h + b1_ref[...], 0.0)
    logits = (
        jnp.dot(h, w2_ref[...], preferred_element_type=jnp.float32) + b2_ref[...]
    )
    out_ref[...] = jnp.concatenate(
        [logits[k * RQ:(k + 1) * RQ, :] for k in range(8)], axis=1)


def _mlp(x, W1, b1, W2p, b2p, half):
    return pl.pallas_call(
        _mlp_body,
        grid=(GRIDH,),
        in_specs=[
            pl.BlockSpec((TPW, D), lambda i: (i + half * GRIDH, 0)),
            pl.BlockSpec((D, D), lambda i: (0, 0)),
            pl.BlockSpec((1, D), lambda i: (0, 0)),
            pl.BlockSpec((D, CP), lambda i: (0, 0)),
            pl.BlockSpec((1, CP), lambda i: (0, 0)),
        ],
        out_specs=pl.BlockSpec((RQ, D), lambda i: (i, 0)),
        out_shape=jax.ShapeDtypeStruct((PRH, D), jnp.float32),
    )(x, W1, b1, W2p, b2p)


def _make_seg_sum(half):
    def _seg_body(log_hbm, bat_hbm, zero_hbm, out_hbm,
                  log_v, idx_v, shacc, sem0, sem1):
        cid = lax.axis_index("c")
        sid = lax.axis_index("s")
        wid = cid * NS + sid
        j = wid // 4
        q = wid % 4

        cp_log = pltpu.async_copy(
            log_hbm.at[pl.ds(q * WR, WR), pl.ds(j * CP, CP)], log_v, sem0)
        bbase = half * NPH + (q // 2) * TPW + j * RQ + (q % 2) * WR
        cp_idx = pltpu.async_copy(bat_hbm.at[pl.ds(bbase, WR)], idx_v, sem1)

        @pl.when(sid == 0)
        def _zero():
            pltpu.sync_copy(zero_hbm, shacc)

        cp_idx.wait()
        plsc.subcore_barrier()
        cp_log.wait()

        pltpu.async_copy(log_v, shacc.at[idx_v], sem1, add=True).wait()

        plsc.subcore_barrier()
        st = S // NS
        pltpu.sync_copy(shacc.at[pl.ds(sid * st, st), :],
                        out_hbm.at[cid, pl.ds(sid * st, st), :])

    return pl.kernel(
        _seg_body,
        out_type=jax.ShapeDtypeStruct((NC, S, CP), jnp.float32),
        mesh=plsc.VectorSubcoreMesh(core_axis_name="c", subcore_axis_name="s"),
        compiler_params=pltpu.CompilerParams(use_tc_tiling_on_sc=False),
        scratch_types=[
            pltpu.VMEM((WR, CP), jnp.float32),
            pltpu.VMEM((WR,), jnp.int32),
            pltpu.VMEM_SHARED((SA, CP), jnp.float32),
            pltpu.SemaphoreType.DMA,
            pltpu.SemaphoreType.DMA,
        ],
    )


_seg_sum_a = _make_seg_sum(0)
_seg_sum_b = _make_seg_sum(1)


def _combine_body(pa_ref, pb_ref, out_ref):
    out_ref[...] = (pa_ref[0] + pa_ref[1]) + (pb_ref[0] + pb_ref[1])


def _combine(pa, pb):
    return pl.pallas_call(
        _combine_body,
        out_shape=jax.ShapeDtypeStruct((S, CP), jnp.float32),
    )(pa, pb)


@jax.jit
def _run(x, batch, W1, b1, W2, b2):
    W2p = jnp.zeros((D, CP), W2.dtype).at[:, :C].set(W2)
    b2p = jnp.zeros((CP,), b2.dtype).at[:C].set(b2)
    bat = jnp.concatenate(
        [batch.astype(jnp.int32), jnp.full((NP - N,), DS, jnp.int32)])
    zero = jnp.zeros((SA, CP), jnp.float32)
    b1r, b2r = b1[None, :], b2p[None, :]
    logits_a = _mlp(x, W1, b1r, W2p, b2r, 0)
    pa = _seg_sum_a(logits_a, bat, zero)
    logits_b = _mlp(x, W1, b1r, W2p, b2r, 1)
    pb = _seg_sum_b(logits_b, bat, zero)
    out = _combine(pa, pb)
    return out[:, :C]


def kernel(x, batch, W1, b1, W2, b2):
    return _run(x, batch, W1, b1, W2, b2)
